# baseline (speedup 1.0000x reference)
import functools

import jax
import jax.numpy as jnp
from jax import lax
from jax.experimental import pallas as pl
from jax.experimental.pallas import tpu as pltpu
from jax.experimental.pallas import tpu_sc as plsc

_PLACEHOLDER = 42
_B, _N, _D = 4, 8192, 1024
_R = _B * _N

_info = plsc.get_sparse_core_info()
_NC, _NS = _info.num_cores, _info.num_subcores
_NW = _NC * _NS
_RPW = _R // _NW
_CH = 32
_NBUF = 3
_ITERS = _RPW // _CH


def _matvec_body(s_ref, w_ref, b_ref, o_ref):
    o_ref[...] = (
        jnp.dot(s_ref[...], w_ref[...], preferred_element_type=jnp.float32)
        + b_ref[...]
    )


@functools.partial(
    pl.kernel,
    out_type=jax.ShapeDtypeStruct((_R, _D), jnp.float32),
    mesh=plsc.VectorSubcoreMesh(core_axis_name="c", subcore_axis_name="s"),
    compiler_params=pltpu.CompilerParams(needs_layout_passes=False),
    scratch_types=[
        pltpu.VMEM((_RPW,), jnp.int32),
        pltpu.VMEM((1, _D), jnp.float32),
        pltpu.VMEM((_NBUF, _CH, _D), jnp.float32),
    ]
    + [pltpu.SemaphoreType.DMA] * (2 * _NBUF),
)
def _sc_body(tok_hbm, x_hbm, rep_hbm, out_hbm, tok_v, rep_v, buf, *sems):
    in_sems, st_sems = sems[:_NBUF], sems[_NBUF:]
    wid = lax.axis_index("s") * _NC + lax.axis_index("c")
    base = wid * _RPW

    pltpu.sync_copy(tok_hbm.at[pl.ds(base, _RPW)], tok_v)
    pltpu.sync_copy(rep_hbm, rep_v)

    def in_copy(it):
        k = it % _NBUF
        return pltpu.make_async_copy(
            x_hbm.at[pl.ds(base + it * _CH, _CH)], buf.at[k], in_sems[k])

    def st_copy(it):
        k = it % _NBUF
        return pltpu.make_async_copy(
            buf.at[k], out_hbm.at[pl.ds(base + it * _CH, _CH)], st_sems[k])

    for k in range(_NBUF):
        in_copy(k).start()
    for it in range(_ITERS):
        in_copy(it).wait()
        st_copy(it).start()
        nxt = it + _NBUF
        if nxt < _ITERS:
            st_copy(it).wait()
            in_copy(nxt).start()
    for it in range(_ITERS - _NBUF, _ITERS):
        st_copy(it).wait()

    zeros = jnp.zeros((16,), jnp.int32)
    ones = jnp.full((16,), 1, jnp.int32)
    ph = jnp.full((16,), _PLACEHOLDER, jnp.int32)

    iota = lax.iota(jnp.int32, 16)
    zeros = jnp.zeros((16,), jnp.int32)
    ones = jnp.full((16,), 1, jnp.int32)
    ph = jnp.full((16,), _PLACEHOLDER, jnp.int32)

    def _group(g, carry):
        tv = tok_v[pl.ds(g * 16, 16)]
        m = jnp.where(tv == ph, ones, zeros)
        cnt = jnp.sum(m)

        @pl.when(cnt > 0)
        def _scan_lanes():
            def _lane(l, c2):
                lb = lax.broadcast_in_dim(l, (16,), ())
                hit = jnp.sum(jnp.where(iota == lb, m, zeros))

                @pl.when(hit > 0)
                def _dma():
                    pltpu.sync_copy(
                        rep_v, out_hbm.at[pl.ds(base + g * 16 + l, 1)])

                return c2

            lax.fori_loop(0, 16, _lane, 0)

        return carry

    lax.fori_loop(0, _RPW // 16, _group, 0)


def kernel(tokenized_text, embedded_text, special_embeddings, W, b):
    B, N, D = embedded_text.shape
    R = B * N
    x = embedded_text.reshape(R, D)
    tok = tokenized_text.reshape(R).astype(jnp.int32)
    s = special_embeddings.reshape(1, D)
    bias = b.reshape(1, D)

    rep = pl.pallas_call(
        _matvec_body,
        out_shape=jax.ShapeDtypeStruct((1, D), jnp.float32),
    )(s, W, bias)

    out = _sc_body(tok, x, rep)
    return out.reshape(B, N, D)
